# baseline (speedup 1.0000x reference)
import jax
import jax.numpy as jnp
from jax import lax
from jax.experimental import pallas as pl
from jax.experimental.pallas import tpu as pltpu
from jax.experimental.pallas import tpu_sc as plsc

_N = 5000
_NPAD = 5120
_L = 16
_NW = 16
_U = 2
_B = _NW * _L * _U
_NB = _NPAD // _B
_FAR = 2e9


def _splat_gather(ref, idx_scalar):
    iv = jnp.full((_L,), idx_scalar, jnp.int32)
    return plsc.load_gather(ref, [iv])


def _nms_body(x1h, y1h, x2h, y2h, keep_h,
              x1, y1, x2, y2,
              kx1, ky1, kx2, ky2, kar,
              stat_l, bidx_l, alive_l, keepv,
              stage, knew_l,
              alive_sh, stat_sh, bidx_sh, knew_sh):
    w = lax.axis_index("s")
    lanes = lax.broadcasted_iota(jnp.int32, (_L,), 0)
    lane0 = lanes == 0
    ffalse = lanes < 0
    fone = jnp.full((_L,), 1.0, jnp.float32)

    pltpu.sync_copy(x1h, x1)
    pltpu.sync_copy(y1h, y1)
    pltpu.sync_copy(x2h, x2)
    pltpu.sync_copy(y2h, y2)

    far = jnp.full((_L,), _FAR, jnp.float32)

    def initk(i, c):
        sl = pl.ds(i * _L, _L)
        kx1[sl] = far
        ky1[sl] = far
        kx2[sl] = far
        ky2[sl] = far
        kar[sl] = fone
        return c
    lax.fori_loop(0, _NPAD // _L, initk, 0)

    @pl.when(w == 0)
    def _():
        def initkv(i, c):
            keepv[pl.ds(i * _L, _L)] = jnp.zeros((_L,), jnp.float32)
            return c
        lax.fori_loop(0, _NPAD // _L, initkv, 0)

    def block_body(jb, k_count):
        base = jb * _B
        mybase = base + w * (_L * _U)

        ca_x1 = x1[pl.ds(mybase, _L)]
        ca_y1 = y1[pl.ds(mybase, _L)]
        ca_x2 = x2[pl.ds(mybase, _L)]
        ca_y2 = y2[pl.ds(mybase, _L)]
        cb_x1 = x1[pl.ds(mybase + _L, _L)]
        cb_y1 = y1[pl.ds(mybase + _L, _L)]
        cb_x2 = x2[pl.ds(mybase + _L, _L)]
        cb_y2 = y2[pl.ds(mybase + _L, _L)]
        ca_ar = (ca_x2 - ca_x1) * (ca_y2 - ca_y1)
        cb_ar = (cb_x2 - cb_x1) * (cb_y2 - cb_y1)

        def scan_one(t, sa, sb):
            kx1v = _splat_gather(kx1, t)
            ky1v = _splat_gather(ky1, t)
            kx2v = _splat_gather(kx2, t)
            ky2v = _splat_gather(ky2, t)
            karv = _splat_gather(kar, t)

            wa = jnp.maximum(jnp.minimum(ca_x2, kx2v) - jnp.maximum(ca_x1, kx1v), 0.0)
            ha = jnp.maximum(jnp.minimum(ca_y2, ky2v) - jnp.maximum(ca_y1, ky1v), 0.0)
            ia = wa * ha
            sa = sa | (ia > 0.5 * (ca_ar + karv - ia))

            wb = jnp.maximum(jnp.minimum(cb_x2, kx2v) - jnp.maximum(cb_x1, kx1v), 0.0)
            hb = jnp.maximum(jnp.minimum(cb_y2, ky2v) - jnp.maximum(cb_y1, ky1v), 0.0)
            ib = wb * hb
            sb = sb | (ib > 0.5 * (cb_ar + karv - ib))
            return sa, sb

        def scan_kept2(t, sup):
            sa, sb = sup
            sa, sb = scan_one(2 * t, sa, sb)
            sa, sb = scan_one(2 * t + 1, sa, sb)
            return sa, sb

        sup_a, sup_b = lax.fori_loop(0, (k_count + 1) // 2, scan_kept2,
                                     (ffalse, ffalse))
        stage[pl.ds(0, _L)] = jnp.where(sup_a, 0, 1).astype(jnp.int32)
        stage[pl.ds(_L, _L)] = jnp.where(sup_b, 0, 1).astype(jnp.int32)
        pltpu.sync_copy(stage, alive_sh.at[pl.ds(w * (_L * _U), _L * _U)])
        plsc.subcore_barrier()

        pltpu.sync_copy(alive_sh, alive_l)
        mypos_a = lanes + w * (_L * _U)
        mypos_b = mypos_a + _L
        basev = jnp.full((_L,), base, jnp.int32)

        def a2_vreg(v, sup2):
            av = alive_l[pl.ds(v * _L, _L)] != 0

            def process(s2):
                def wbody(carry):
                    s2a, s2b, m = carry
                    iv = plsc.all_reduce_ffs(m)
                    qpos = jnp.full((_L,), v * _L, jnp.int32) + iv
                    m2 = m & (lanes != iv)
                    gq = basev + qpos
                    qx1 = plsc.load_gather(x1, [gq])
                    qy1 = plsc.load_gather(y1, [gq])
                    qx2 = plsc.load_gather(x2, [gq])
                    qy2 = plsc.load_gather(y2, [gq])
                    qar = (qx2 - qx1) * (qy2 - qy1)

                    wa = jnp.maximum(jnp.minimum(ca_x2, qx2) - jnp.maximum(ca_x1, qx1), 0.0)
                    ha = jnp.maximum(jnp.minimum(ca_y2, qy2) - jnp.maximum(ca_y1, qy1), 0.0)
                    ia = wa * ha
                    s2a = s2a | ((ia > 0.5 * (ca_ar + qar - ia)) & (qpos < mypos_a))

                    wb = jnp.maximum(jnp.minimum(cb_x2, qx2) - jnp.maximum(cb_x1, qx1), 0.0)
                    hb = jnp.maximum(jnp.minimum(cb_y2, qy2) - jnp.maximum(cb_y1, qy1), 0.0)
                    ib = wb * hb
                    s2b = s2b | ((ib > 0.5 * (cb_ar + qar - ib)) & (qpos < mypos_b))
                    return (s2a, s2b, m2)

                s2a, s2b, _ = lax.while_loop(lambda c: jnp.any(c[2]), wbody,
                                             (s2[0], s2[1], av))
                return (s2a, s2b)

            return lax.cond(jnp.any(av), process, lambda s: s, sup2)

        sup2_a, sup2_b = lax.fori_loop(0, _B // _L, a2_vreg, (ffalse, ffalse))
        stat_a = jnp.where(sup_a, 0, jnp.where(sup2_a, 2, 1)).astype(jnp.int32)
        stat_b = jnp.where(sup_b, 0, jnp.where(sup2_b, 2, 1)).astype(jnp.int32)
        stage[pl.ds(0, _L)] = stat_a
        stage[pl.ds(_L, _L)] = stat_b
        pltpu.sync_copy(stage, stat_sh.at[pl.ds(w * (_L * _U), _L * _U)])
        plsc.subcore_barrier()

        @pl.when(w == 0)
        def _():
            pltpu.sync_copy(stat_sh, stat_l)

            def initb(i, c):
                bidx_l[pl.ds(i * _L, _L)] = jnp.zeros((_L,), jnp.int32)
                return c
            lax.fori_loop(0, _B // _L, initb, 0)

            def res_vreg(v, c0):
                sv = stat_l[pl.ds(v * _L, _L)]
                um = sv == 2

                def process(c1):
                    def wbody(carry):
                        _, m = carry
                        iv = plsc.all_reduce_ffs(m)
                        m2 = m & (lanes != iv)
                        pv = jnp.full((_L,), v * _L, jnp.int32) + iv
                        gv = basev + pv
                        cx1 = plsc.load_gather(x1, [gv])
                        cy1 = plsc.load_gather(y1, [gv])
                        cx2 = plsc.load_gather(x2, [gv])
                        cy2 = plsc.load_gather(y2, [gv])
                        car = (cx2 - cx1) * (cy2 - cy1)

                        def inner(t, sup):
                            sl = pl.ds(base + t * _L, _L)
                            sv2 = stat_l[pl.ds(t * _L, _L)]
                            posv = lanes + t * _L
                            ex1 = x1[sl]
                            ey1 = y1[sl]
                            ex2 = x2[sl]
                            ey2 = y2[sl]
                            ear = (ex2 - ex1) * (ey2 - ey1)
                            wv = jnp.maximum(jnp.minimum(cx2, ex2)
                                             - jnp.maximum(cx1, ex1), 0.0)
                            hv = jnp.maximum(jnp.minimum(cy2, ey2)
                                             - jnp.maximum(cy1, ey1), 0.0)
                            ivr = wv * hv
                            hit = (ivr > 0.5 * (car + ear - ivr))
                            return sup | (hit & (sv2 == 1) & (posv < pv))

                        sup = lax.fori_loop(0, v + 1, inner, ffalse)
                        supi = jnp.any(sup).astype(jnp.int32)
                        val = jnp.full((_L,), 1, jnp.int32) - supi
                        plsc.store_scatter(stat_l, [pv], val, mask=lane0)
                        return (c1, m2)

                    return lax.while_loop(lambda c: jnp.any(c[1]), wbody,
                                          (c1, um))[0]

                return lax.cond(jnp.any(um), process, lambda c: c, c0)

            lax.fori_loop(0, _B // _L, res_vreg, 0)

            def cmp_vreg(v, nb0):
                sv = stat_l[pl.ds(v * _L, _L)]
                km = sv == 1

                def process(nb1):
                    def wbody(carry):
                        nbc, m = carry
                        iv = plsc.all_reduce_ffs(m)
                        m2 = m & (lanes != iv)
                        pv = jnp.full((_L,), v * _L, jnp.int32) + iv
                        nv = jnp.full((_L,), nbc, jnp.int32)
                        plsc.store_scatter(bidx_l, [nv], pv, mask=lane0)
                        plsc.store_scatter(keepv, [basev + pv], fone, mask=lane0)
                        return (nbc + 1, m2)

                    return lax.while_loop(lambda c: jnp.any(c[1]), wbody,
                                          (nb1, km))[0]

                return lax.cond(jnp.any(km), process, lambda n: n, nb0)

            bk = lax.fori_loop(0, _B // _L, cmp_vreg, 0)
            pltpu.sync_copy(bidx_l, bidx_sh)
            stage[pl.ds(0, _L)] = jnp.full((_L,), bk, jnp.int32)
            pltpu.sync_copy(stage.at[pl.ds(0, _L)], knew_sh)

        plsc.subcore_barrier()

        pltpu.sync_copy(knew_sh, knew_l)
        nb = jnp.max(knew_l[pl.ds(0, _L)])
        pltpu.sync_copy(bidx_sh, bidx_l)

        def append_delta(t, c):
            idxv = basev + bidx_l[pl.ds(t * _L, _L)]
            m = (lanes + t * _L) < nb
            kvec = lanes + (k_count + t * _L)
            gx1 = plsc.load_gather(x1, [idxv])
            gy1 = plsc.load_gather(y1, [idxv])
            gx2 = plsc.load_gather(x2, [idxv])
            gy2 = plsc.load_gather(y2, [idxv])
            plsc.store_scatter(kx1, [kvec], gx1, mask=m)
            plsc.store_scatter(ky1, [kvec], gy1, mask=m)
            plsc.store_scatter(kx2, [kvec], gx2, mask=m)
            plsc.store_scatter(ky2, [kvec], gy2, mask=m)
            plsc.store_scatter(kar, [kvec], (gx2 - gx1) * (gy2 - gy1), mask=m)
            return c

        lax.fori_loop(0, (nb + _L - 1) // _L, append_delta, 0)
        return k_count + nb

    lax.fori_loop(0, _NB, block_body, 0)

    @pl.when(w == 0)
    def _():
        pltpu.sync_copy(keepv, keep_h)


@jax.jit
def _nms_keep01(x1s, y1s, x2s, y2s):
    mesh = plsc.VectorSubcoreMesh(core_axis_name="c", subcore_axis_name="s",
                                  num_cores=1)
    f = pl.kernel(
        _nms_body,
        out_type=jax.ShapeDtypeStruct((_NPAD,), jnp.float32),
        mesh=mesh,
        scratch_types=(
            [pltpu.VMEM((_NPAD,), jnp.float32) for _ in range(4)]
            + [pltpu.VMEM((_NPAD,), jnp.float32) for _ in range(5)]
            + [pltpu.VMEM((_B,), jnp.int32)]
            + [pltpu.VMEM((_B,), jnp.int32)]
            + [pltpu.VMEM((_B,), jnp.int32)]
            + [pltpu.VMEM((_NPAD,), jnp.float32)]
            + [pltpu.VMEM((_L * _U,), jnp.int32)]
            + [pltpu.VMEM((_L,), jnp.int32)]
            + [pltpu.VMEM_SHARED((_B,), jnp.int32)]
            + [pltpu.VMEM_SHARED((_B,), jnp.int32)]
            + [pltpu.VMEM_SHARED((_B,), jnp.int32)]
            + [pltpu.VMEM_SHARED((_L,), jnp.int32)]
        ),
        compiler_params=pltpu.CompilerParams(needs_layout_passes=False),
    )
    return f(x1s, y1s, x2s, y2s)


def kernel(boxes, scores):
    order = jnp.argsort(-scores)
    boxes_sorted = boxes[order]
    scores_sorted = scores[order]
    pad = _NPAD - boxes_sorted.shape[0]
    bp = jnp.concatenate(
        [boxes_sorted, jnp.broadcast_to(boxes_sorted[0], (pad, 4))], axis=0)
    keep01 = _nms_keep01(bp[:, 0], bp[:, 1], bp[:, 2], bp[:, 3])[:_N]
    keep = keep01 > 0.5
    kept_scores = scores_sorted * keep01
    return kept_scores, keep, order


# speedup vs baseline: 1.1530x; 1.1530x over previous
import jax
import jax.numpy as jnp
from jax import lax
from jax.experimental import pallas as pl
from jax.experimental.pallas import tpu as pltpu
from jax.experimental.pallas import tpu_sc as plsc

_N = 5000
_NPAD = 5120
_L = 16
_NW = 16
_U = 2
_B = _NW * _L * _U
_NB = _NPAD // _B
_FAR = 2e9


def _splat_gather(ref, idx_scalar):
    iv = jnp.full((_L,), idx_scalar, jnp.int32)
    return plsc.load_gather(ref, [iv])


def _nms_body(x1h, y1h, x2h, y2h, keep_h,
              x1, y1, x2, y2,
              kx1, ky1, kx2, ky2, kar,
              stat_l, bidx_l, alive_l, keepv,
              stage, knew_l,
              alive_sh, stat_sh, bidx_sh, knew_sh):
    w = lax.axis_index("s")
    lanes = lax.broadcasted_iota(jnp.int32, (_L,), 0)
    lane0 = lanes == 0
    ffalse = lanes < 0
    fone = jnp.full((_L,), 1.0, jnp.float32)

    pltpu.sync_copy(x1h, x1)
    pltpu.sync_copy(y1h, y1)
    pltpu.sync_copy(x2h, x2)
    pltpu.sync_copy(y2h, y2)

    far = jnp.full((_L,), _FAR, jnp.float32)

    def initk(i, c):
        sl = pl.ds(i * _L, _L)
        kx1[sl] = far
        ky1[sl] = far
        kx2[sl] = far
        ky2[sl] = far
        kar[sl] = fone
        return c
    lax.fori_loop(0, _NPAD // _L, initk, 0)

    @pl.when(w == 0)
    def _():
        def initkv(i, c):
            keepv[pl.ds(i * _L, _L)] = jnp.zeros((_L,), jnp.float32)
            return c
        lax.fori_loop(0, _NPAD // _L, initkv, 0)

    p1 = w * _L
    p2 = (2 * _NW - 1 - w) * _L

    def block_body(jb, k_count):
        base = jb * _B

        ca_x1 = x1[pl.ds(base + p1, _L)]
        ca_y1 = y1[pl.ds(base + p1, _L)]
        ca_x2 = x2[pl.ds(base + p1, _L)]
        ca_y2 = y2[pl.ds(base + p1, _L)]
        cb_x1 = x1[pl.ds(base + p2, _L)]
        cb_y1 = y1[pl.ds(base + p2, _L)]
        cb_x2 = x2[pl.ds(base + p2, _L)]
        cb_y2 = y2[pl.ds(base + p2, _L)]
        ca_ar = (ca_x2 - ca_x1) * (ca_y2 - ca_y1)
        cb_ar = (cb_x2 - cb_x1) * (cb_y2 - cb_y1)

        def scan_one(t, sa, sb):
            kx1v = _splat_gather(kx1, t)
            ky1v = _splat_gather(ky1, t)
            kx2v = _splat_gather(kx2, t)
            ky2v = _splat_gather(ky2, t)
            karv = _splat_gather(kar, t)

            wa = jnp.maximum(jnp.minimum(ca_x2, kx2v) - jnp.maximum(ca_x1, kx1v), 0.0)
            ha = jnp.maximum(jnp.minimum(ca_y2, ky2v) - jnp.maximum(ca_y1, ky1v), 0.0)
            ia = wa * ha
            sa = sa | (ia > 0.5 * (ca_ar + karv - ia))

            wb = jnp.maximum(jnp.minimum(cb_x2, kx2v) - jnp.maximum(cb_x1, kx1v), 0.0)
            hb = jnp.maximum(jnp.minimum(cb_y2, ky2v) - jnp.maximum(cb_y1, ky1v), 0.0)
            ib = wb * hb
            sb = sb | (ib > 0.5 * (cb_ar + karv - ib))
            return sa, sb

        def scan_kept2(t, sup):
            sa, sb = sup
            sa, sb = scan_one(2 * t, sa, sb)
            sa, sb = scan_one(2 * t + 1, sa, sb)
            return sa, sb

        sup_a, sup_b = lax.fori_loop(0, (k_count + 1) // 2, scan_kept2,
                                     (ffalse, ffalse))
        stage[pl.ds(0, _L)] = jnp.where(sup_a, 0, 1).astype(jnp.int32)
        stage[pl.ds(_L, _L)] = jnp.where(sup_b, 0, 1).astype(jnp.int32)
        pltpu.sync_copy(stage.at[pl.ds(0, _L)], alive_sh.at[pl.ds(p1, _L)])
        pltpu.sync_copy(stage.at[pl.ds(_L, _L)], alive_sh.at[pl.ds(p2, _L)])
        plsc.subcore_barrier()

        pltpu.sync_copy(alive_sh, alive_l)
        mypos_a = lanes + p1
        mypos_b = lanes + p2
        basev = jnp.full((_L,), base, jnp.int32)

        def a2_scan(cx1, cy1, cx2, cy2, car, mypos):
            def step(t, s2):
                alive_q = _splat_gather(alive_l, t) != 0
                qpos = jnp.full((_L,), t, jnp.int32)
                gq = basev + qpos
                qx1 = plsc.load_gather(x1, [gq])
                qy1 = plsc.load_gather(y1, [gq])
                qx2 = plsc.load_gather(x2, [gq])
                qy2 = plsc.load_gather(y2, [gq])
                qar = (qx2 - qx1) * (qy2 - qy1)
                wv = jnp.maximum(jnp.minimum(cx2, qx2) - jnp.maximum(cx1, qx1), 0.0)
                hv = jnp.maximum(jnp.minimum(cy2, qy2) - jnp.maximum(cy1, qy1), 0.0)
                iv = wv * hv
                return s2 | ((iv > 0.5 * (car + qar - iv)) & alive_q
                             & (qpos < mypos))
            return step

        sup2_a = lax.fori_loop(0, p1 + _L,
                               a2_scan(ca_x1, ca_y1, ca_x2, ca_y2, ca_ar,
                                       mypos_a), ffalse)
        sup2_b = lax.fori_loop(0, p2 + _L,
                               a2_scan(cb_x1, cb_y1, cb_x2, cb_y2, cb_ar,
                                       mypos_b), ffalse)
        stat_a = jnp.where(sup_a, 0, jnp.where(sup2_a, 2, 1)).astype(jnp.int32)
        stat_b = jnp.where(sup_b, 0, jnp.where(sup2_b, 2, 1)).astype(jnp.int32)
        stage[pl.ds(0, _L)] = stat_a
        stage[pl.ds(_L, _L)] = stat_b
        pltpu.sync_copy(stage.at[pl.ds(0, _L)], stat_sh.at[pl.ds(p1, _L)])
        pltpu.sync_copy(stage.at[pl.ds(_L, _L)], stat_sh.at[pl.ds(p2, _L)])
        plsc.subcore_barrier()

        @pl.when(w == 0)
        def _():
            pltpu.sync_copy(stat_sh, stat_l)

            def initb(i, c):
                bidx_l[pl.ds(i * _L, _L)] = jnp.zeros((_L,), jnp.int32)
                return c
            lax.fori_loop(0, _B // _L, initb, 0)

            def res_vreg(v, c0):
                sv = stat_l[pl.ds(v * _L, _L)]
                um = sv == 2

                def process(c1):
                    def wbody(carry):
                        _, m = carry
                        iv = plsc.all_reduce_ffs(m)
                        m2 = m & (lanes != iv)
                        pv = jnp.full((_L,), v * _L, jnp.int32) + iv
                        gv = basev + pv
                        cx1 = plsc.load_gather(x1, [gv])
                        cy1 = plsc.load_gather(y1, [gv])
                        cx2 = plsc.load_gather(x2, [gv])
                        cy2 = plsc.load_gather(y2, [gv])
                        car = (cx2 - cx1) * (cy2 - cy1)

                        def inner(t, sup):
                            sl = pl.ds(base + t * _L, _L)
                            sv2 = stat_l[pl.ds(t * _L, _L)]
                            posv = lanes + t * _L
                            ex1 = x1[sl]
                            ey1 = y1[sl]
                            ex2 = x2[sl]
                            ey2 = y2[sl]
                            ear = (ex2 - ex1) * (ey2 - ey1)
                            wv = jnp.maximum(jnp.minimum(cx2, ex2)
                                             - jnp.maximum(cx1, ex1), 0.0)
                            hv = jnp.maximum(jnp.minimum(cy2, ey2)
                                             - jnp.maximum(cy1, ey1), 0.0)
                            ivr = wv * hv
                            hit = (ivr > 0.5 * (car + ear - ivr))
                            return sup | (hit & (sv2 == 1) & (posv < pv))

                        sup = lax.fori_loop(0, v + 1, inner, ffalse)
                        supi = jnp.any(sup).astype(jnp.int32)
                        val = jnp.full((_L,), 1, jnp.int32) - supi
                        plsc.store_scatter(stat_l, [pv], val, mask=lane0)
                        return (c1, m2)

                    return lax.while_loop(lambda c: jnp.any(c[1]), wbody,
                                          (c1, um))[0]

                return lax.cond(jnp.any(um), process, lambda c: c, c0)

            lax.fori_loop(0, _B // _L, res_vreg, 0)

            def cmp_vreg(v, nb0):
                sv = stat_l[pl.ds(v * _L, _L)]
                km = sv == 1

                def process(nb1):
                    def wbody(carry):
                        nbc, m = carry
                        iv = plsc.all_reduce_ffs(m)
                        m2 = m & (lanes != iv)
                        pv = jnp.full((_L,), v * _L, jnp.int32) + iv
                        nv = jnp.full((_L,), nbc, jnp.int32)
                        plsc.store_scatter(bidx_l, [nv], pv, mask=lane0)
                        plsc.store_scatter(keepv, [basev + pv], fone, mask=lane0)
                        return (nbc + 1, m2)

                    return lax.while_loop(lambda c: jnp.any(c[1]), wbody,
                                          (nb1, km))[0]

                return lax.cond(jnp.any(km), process, lambda n: n, nb0)

            bk = lax.fori_loop(0, _B // _L, cmp_vreg, 0)
            pltpu.sync_copy(bidx_l, bidx_sh)
            stage[pl.ds(0, _L)] = jnp.full((_L,), bk, jnp.int32)
            pltpu.sync_copy(stage.at[pl.ds(0, _L)], knew_sh)

        plsc.subcore_barrier()

        pltpu.sync_copy(knew_sh, knew_l)
        nb = jnp.max(knew_l[pl.ds(0, _L)])
        pltpu.sync_copy(bidx_sh, bidx_l)

        def append_delta(t, c):
            idxv = basev + bidx_l[pl.ds(t * _L, _L)]
            m = (lanes + t * _L) < nb
            kvec = lanes + (k_count + t * _L)
            gx1 = plsc.load_gather(x1, [idxv])
            gy1 = plsc.load_gather(y1, [idxv])
            gx2 = plsc.load_gather(x2, [idxv])
            gy2 = plsc.load_gather(y2, [idxv])
            plsc.store_scatter(kx1, [kvec], gx1, mask=m)
            plsc.store_scatter(ky1, [kvec], gy1, mask=m)
            plsc.store_scatter(kx2, [kvec], gx2, mask=m)
            plsc.store_scatter(ky2, [kvec], gy2, mask=m)
            plsc.store_scatter(kar, [kvec], (gx2 - gx1) * (gy2 - gy1), mask=m)
            return c

        lax.fori_loop(0, (nb + _L - 1) // _L, append_delta, 0)
        return k_count + nb

    lax.fori_loop(0, _NB, block_body, 0)

    @pl.when(w == 0)
    def _():
        pltpu.sync_copy(keepv, keep_h)


@jax.jit
def _nms_keep01(x1s, y1s, x2s, y2s):
    mesh = plsc.VectorSubcoreMesh(core_axis_name="c", subcore_axis_name="s",
                                  num_cores=1)
    f = pl.kernel(
        _nms_body,
        out_type=jax.ShapeDtypeStruct((_NPAD,), jnp.float32),
        mesh=mesh,
        scratch_types=(
            [pltpu.VMEM((_NPAD,), jnp.float32) for _ in range(4)]
            + [pltpu.VMEM((_NPAD,), jnp.float32) for _ in range(5)]
            + [pltpu.VMEM((_B,), jnp.int32)]
            + [pltpu.VMEM((_B,), jnp.int32)]
            + [pltpu.VMEM((_B,), jnp.int32)]
            + [pltpu.VMEM((_NPAD,), jnp.float32)]
            + [pltpu.VMEM((_L * _U,), jnp.int32)]
            + [pltpu.VMEM((_L,), jnp.int32)]
            + [pltpu.VMEM_SHARED((_B,), jnp.int32)]
            + [pltpu.VMEM_SHARED((_B,), jnp.int32)]
            + [pltpu.VMEM_SHARED((_B,), jnp.int32)]
            + [pltpu.VMEM_SHARED((_L,), jnp.int32)]
        ),
        compiler_params=pltpu.CompilerParams(needs_layout_passes=False),
    )
    return f(x1s, y1s, x2s, y2s)


def kernel(boxes, scores):
    order = jnp.argsort(-scores)
    boxes_sorted = boxes[order]
    scores_sorted = scores[order]
    pad = _NPAD - boxes_sorted.shape[0]
    bp = jnp.concatenate(
        [boxes_sorted, jnp.broadcast_to(boxes_sorted[0], (pad, 4))], axis=0)
    keep01 = _nms_keep01(bp[:, 0], bp[:, 1], bp[:, 2], bp[:, 3])[:_N]
    keep = keep01 > 0.5
    kept_scores = scores_sorted * keep01
    return kept_scores, keep, order


# speedup vs baseline: 1.1816x; 1.0248x over previous
import jax
import jax.numpy as jnp
from jax import lax
from jax.experimental import pallas as pl
from jax.experimental.pallas import tpu as pltpu
from jax.experimental.pallas import tpu_sc as plsc

_N = 5000
_NPAD = 5120
_L = 16
_NW = 16
_U = 2
_B = _NW * _L * _U
_NB = _NPAD // _B
_FAR = 2e9


def _splat_gather(ref, idx_scalar):
    iv = jnp.full((_L,), idx_scalar, jnp.int32)
    return plsc.load_gather(ref, [iv])


def _nms_body(x1h, y1h, x2h, y2h, sch, ordh, keep_h, ksc_h,
              x1, y1, x2, y2, scu, ordl,
              kx1, ky1, kx2, ky2, kar,
              stat_l, bidx_l, alive_l, keepv, ksc_v,
              stage, knew_l,
              alive_sh, stat_sh, bidx_sh, knew_sh):
    w = lax.axis_index("s")
    lanes = lax.broadcasted_iota(jnp.int32, (_L,), 0)
    lane0 = lanes == 0
    ffalse = lanes < 0
    fone = jnp.full((_L,), 1.0, jnp.float32)

    pltpu.sync_copy(x1h, x1)
    pltpu.sync_copy(y1h, y1)
    pltpu.sync_copy(x2h, x2)
    pltpu.sync_copy(y2h, y2)
    pltpu.sync_copy(sch, scu)
    pltpu.sync_copy(ordh, ordl)

    far = jnp.full((_L,), _FAR, jnp.float32)

    def initk(i, c):
        sl = pl.ds(i * _L, _L)
        kx1[sl] = far
        ky1[sl] = far
        kx2[sl] = far
        ky2[sl] = far
        kar[sl] = fone
        return c
    lax.fori_loop(0, _NPAD // _L, initk, 0)

    @pl.when(w == 0)
    def _():
        def initkv(i, c):
            keepv[pl.ds(i * _L, _L)] = jnp.zeros((_L,), jnp.float32)
            ksc_v[pl.ds(i * _L, _L)] = jnp.zeros((_L,), jnp.float32)
            return c
        lax.fori_loop(0, _NPAD // _L, initkv, 0)

    p1 = w * _L
    p2 = (2 * _NW - 1 - w) * _L

    def block_body(jb, k_count):
        base = jb * _B

        orda = ordl[pl.ds(base + p1, _L)]
        ordb = ordl[pl.ds(base + p2, _L)]
        ca_x1 = plsc.load_gather(x1, [orda])
        ca_y1 = plsc.load_gather(y1, [orda])
        ca_x2 = plsc.load_gather(x2, [orda])
        ca_y2 = plsc.load_gather(y2, [orda])
        cb_x1 = plsc.load_gather(x1, [ordb])
        cb_y1 = plsc.load_gather(y1, [ordb])
        cb_x2 = plsc.load_gather(x2, [ordb])
        cb_y2 = plsc.load_gather(y2, [ordb])
        ca_ar = (ca_x2 - ca_x1) * (ca_y2 - ca_y1)
        cb_ar = (cb_x2 - cb_x1) * (cb_y2 - cb_y1)

        def scan_one(t, sa, sb):
            kx1v = _splat_gather(kx1, t)
            ky1v = _splat_gather(ky1, t)
            kx2v = _splat_gather(kx2, t)
            ky2v = _splat_gather(ky2, t)
            karv = _splat_gather(kar, t)

            wa = jnp.maximum(jnp.minimum(ca_x2, kx2v) - jnp.maximum(ca_x1, kx1v), 0.0)
            ha = jnp.maximum(jnp.minimum(ca_y2, ky2v) - jnp.maximum(ca_y1, ky1v), 0.0)
            ia = wa * ha
            sa = sa | (ia > 0.5 * (ca_ar + karv - ia))

            wb = jnp.maximum(jnp.minimum(cb_x2, kx2v) - jnp.maximum(cb_x1, kx1v), 0.0)
            hb = jnp.maximum(jnp.minimum(cb_y2, ky2v) - jnp.maximum(cb_y1, ky1v), 0.0)
            ib = wb * hb
            sb = sb | (ib > 0.5 * (cb_ar + karv - ib))
            return sa, sb

        def scan_kept2(t, sup):
            sa, sb = sup
            sa, sb = scan_one(2 * t, sa, sb)
            sa, sb = scan_one(2 * t + 1, sa, sb)
            return sa, sb

        sup_a, sup_b = lax.fori_loop(0, (k_count + 1) // 2, scan_kept2,
                                     (ffalse, ffalse))
        stage[pl.ds(0, _L)] = jnp.where(sup_a, 0, 1).astype(jnp.int32)
        stage[pl.ds(_L, _L)] = jnp.where(sup_b, 0, 1).astype(jnp.int32)
        pltpu.sync_copy(stage.at[pl.ds(0, _L)], alive_sh.at[pl.ds(p1, _L)])
        pltpu.sync_copy(stage.at[pl.ds(_L, _L)], alive_sh.at[pl.ds(p2, _L)])
        plsc.subcore_barrier()

        pltpu.sync_copy(alive_sh, alive_l)
        mypos_a = lanes + p1
        mypos_b = lanes + p2
        basev = jnp.full((_L,), base, jnp.int32)

        def a2_scan(cx1, cy1, cx2, cy2, car, mypos):
            def step(t, s2):
                alive_q = _splat_gather(alive_l, t) != 0
                qpos = jnp.full((_L,), t, jnp.int32)
                oq = plsc.load_gather(ordl, [basev + qpos])
                qx1 = plsc.load_gather(x1, [oq])
                qy1 = plsc.load_gather(y1, [oq])
                qx2 = plsc.load_gather(x2, [oq])
                qy2 = plsc.load_gather(y2, [oq])
                qar = (qx2 - qx1) * (qy2 - qy1)
                wv = jnp.maximum(jnp.minimum(cx2, qx2) - jnp.maximum(cx1, qx1), 0.0)
                hv = jnp.maximum(jnp.minimum(cy2, qy2) - jnp.maximum(cy1, qy1), 0.0)
                iv = wv * hv
                return s2 | ((iv > 0.5 * (car + qar - iv)) & alive_q
                             & (qpos < mypos))
            return step

        sup2_a = lax.fori_loop(0, p1 + _L,
                               a2_scan(ca_x1, ca_y1, ca_x2, ca_y2, ca_ar,
                                       mypos_a), ffalse)
        sup2_b = lax.fori_loop(0, p2 + _L,
                               a2_scan(cb_x1, cb_y1, cb_x2, cb_y2, cb_ar,
                                       mypos_b), ffalse)
        stat_a = jnp.where(sup_a, 0, jnp.where(sup2_a, 2, 1)).astype(jnp.int32)
        stat_b = jnp.where(sup_b, 0, jnp.where(sup2_b, 2, 1)).astype(jnp.int32)
        stage[pl.ds(0, _L)] = stat_a
        stage[pl.ds(_L, _L)] = stat_b
        pltpu.sync_copy(stage.at[pl.ds(0, _L)], stat_sh.at[pl.ds(p1, _L)])
        pltpu.sync_copy(stage.at[pl.ds(_L, _L)], stat_sh.at[pl.ds(p2, _L)])
        plsc.subcore_barrier()

        @pl.when(w == 0)
        def _():
            pltpu.sync_copy(stat_sh, stat_l)

            def initb(i, c):
                bidx_l[pl.ds(i * _L, _L)] = jnp.zeros((_L,), jnp.int32)
                return c
            lax.fori_loop(0, _B // _L, initb, 0)

            def res_vreg(v, c0):
                sv = stat_l[pl.ds(v * _L, _L)]
                um = sv == 2

                def process(c1):
                    def wbody(carry):
                        _, m = carry
                        iv = plsc.all_reduce_ffs(m)
                        m2 = m & (lanes != iv)
                        pv = jnp.full((_L,), v * _L, jnp.int32) + iv
                        gv = basev + pv
                        ov = plsc.load_gather(ordl, [gv])
                        cx1 = plsc.load_gather(x1, [ov])
                        cy1 = plsc.load_gather(y1, [ov])
                        cx2 = plsc.load_gather(x2, [ov])
                        cy2 = plsc.load_gather(y2, [ov])
                        car = (cx2 - cx1) * (cy2 - cy1)

                        def inner(t, sup):
                            sl = pl.ds(base + t * _L, _L)
                            sv2 = stat_l[pl.ds(t * _L, _L)]
                            posv = lanes + t * _L
                            ordv2 = ordl[sl]
                            ex1 = plsc.load_gather(x1, [ordv2])
                            ey1 = plsc.load_gather(y1, [ordv2])
                            ex2 = plsc.load_gather(x2, [ordv2])
                            ey2 = plsc.load_gather(y2, [ordv2])
                            ear = (ex2 - ex1) * (ey2 - ey1)
                            wv = jnp.maximum(jnp.minimum(cx2, ex2)
                                             - jnp.maximum(cx1, ex1), 0.0)
                            hv = jnp.maximum(jnp.minimum(cy2, ey2)
                                             - jnp.maximum(cy1, ey1), 0.0)
                            ivr = wv * hv
                            hit = (ivr > 0.5 * (car + ear - ivr))
                            return sup | (hit & (sv2 == 1) & (posv < pv))

                        sup = lax.fori_loop(0, v + 1, inner, ffalse)
                        supi = jnp.any(sup).astype(jnp.int32)
                        val = jnp.full((_L,), 1, jnp.int32) - supi
                        plsc.store_scatter(stat_l, [pv], val, mask=lane0)
                        return (c1, m2)

                    return lax.while_loop(lambda c: jnp.any(c[1]), wbody,
                                          (c1, um))[0]

                return lax.cond(jnp.any(um), process, lambda c: c, c0)

            lax.fori_loop(0, _B // _L, res_vreg, 0)

            def cmp_vreg(v, nb0):
                sv = stat_l[pl.ds(v * _L, _L)]
                km = sv == 1

                def process(nb1):
                    def wbody(carry):
                        nbc, m = carry
                        iv = plsc.all_reduce_ffs(m)
                        m2 = m & (lanes != iv)
                        pv = jnp.full((_L,), v * _L, jnp.int32) + iv
                        nv = jnp.full((_L,), nbc, jnp.int32)
                        plsc.store_scatter(bidx_l, [nv], pv, mask=lane0)
                        plsc.store_scatter(keepv, [basev + pv], fone, mask=lane0)
                        return (nbc + 1, m2)

                    return lax.while_loop(lambda c: jnp.any(c[1]), wbody,
                                          (nb1, km))[0]

                return lax.cond(jnp.any(km), process, lambda n: n, nb0)

            bk = lax.fori_loop(0, _B // _L, cmp_vreg, 0)
            pltpu.sync_copy(bidx_l, bidx_sh)
            stage[pl.ds(0, _L)] = jnp.full((_L,), bk, jnp.int32)
            pltpu.sync_copy(stage.at[pl.ds(0, _L)], knew_sh)

        plsc.subcore_barrier()

        pltpu.sync_copy(knew_sh, knew_l)
        nb = jnp.max(knew_l[pl.ds(0, _L)])
        pltpu.sync_copy(bidx_sh, bidx_l)

        def append_delta(t, c):
            idxv = basev + bidx_l[pl.ds(t * _L, _L)]
            m = (lanes + t * _L) < nb
            kvec = lanes + (k_count + t * _L)
            ov = plsc.load_gather(ordl, [idxv])
            gx1 = plsc.load_gather(x1, [ov])
            gy1 = plsc.load_gather(y1, [ov])
            gx2 = plsc.load_gather(x2, [ov])
            gy2 = plsc.load_gather(y2, [ov])
            sv = plsc.load_gather(scu, [ov])
            plsc.store_scatter(ksc_v, [idxv], sv, mask=m)
            plsc.store_scatter(kx1, [kvec], gx1, mask=m)
            plsc.store_scatter(ky1, [kvec], gy1, mask=m)
            plsc.store_scatter(kx2, [kvec], gx2, mask=m)
            plsc.store_scatter(ky2, [kvec], gy2, mask=m)
            plsc.store_scatter(kar, [kvec], (gx2 - gx1) * (gy2 - gy1), mask=m)
            return c

        lax.fori_loop(0, (nb + _L - 1) // _L, append_delta, 0)
        return k_count + nb

    lax.fori_loop(0, _NB, block_body, 0)

    @pl.when(w == 0)
    def _():
        pltpu.sync_copy(keepv, keep_h)
        pltpu.sync_copy(ksc_v, ksc_h)


@jax.jit
def _nms_run(x1u, y1u, x2u, y2u, sc, orderp):
    mesh = plsc.VectorSubcoreMesh(core_axis_name="c", subcore_axis_name="s",
                                  num_cores=1)
    f = pl.kernel(
        _nms_body,
        out_type=(jax.ShapeDtypeStruct((_NPAD,), jnp.float32),
                  jax.ShapeDtypeStruct((_NPAD,), jnp.float32)),
        mesh=mesh,
        scratch_types=(
            [pltpu.VMEM((_NPAD,), jnp.float32) for _ in range(4)]
            + [pltpu.VMEM((_NPAD,), jnp.float32)]
            + [pltpu.VMEM((_NPAD,), jnp.int32)]
            + [pltpu.VMEM((_NPAD,), jnp.float32) for _ in range(5)]
            + [pltpu.VMEM((_B,), jnp.int32)]
            + [pltpu.VMEM((_B,), jnp.int32)]
            + [pltpu.VMEM((_B,), jnp.int32)]
            + [pltpu.VMEM((_NPAD,), jnp.float32)]
            + [pltpu.VMEM((_NPAD,), jnp.float32)]
            + [pltpu.VMEM((_L * _U,), jnp.int32)]
            + [pltpu.VMEM((_L,), jnp.int32)]
            + [pltpu.VMEM_SHARED((_B,), jnp.int32)]
            + [pltpu.VMEM_SHARED((_B,), jnp.int32)]
            + [pltpu.VMEM_SHARED((_B,), jnp.int32)]
            + [pltpu.VMEM_SHARED((_L,), jnp.int32)]
        ),
        compiler_params=pltpu.CompilerParams(needs_layout_passes=False),
    )
    return f(x1u, y1u, x2u, y2u, sc, orderp)


def kernel(boxes, scores):
    order = jnp.argsort(-scores)
    pad = _NPAD - boxes.shape[0]
    orderp = jnp.concatenate(
        [order, jnp.full((pad,), order[0], order.dtype)], axis=0)
    bp = jnp.pad(boxes, ((0, pad), (0, 0)))
    sp = jnp.pad(scores, (0, pad))
    keep01, ksc = _nms_run(bp[:, 0], bp[:, 1], bp[:, 2], bp[:, 3], sp, orderp)
    keep = keep01[:_N] > 0.5
    kept_scores = ksc[:_N]
    return kept_scores, keep, order


# speedup vs baseline: 1.1876x; 1.0051x over previous
import jax
import jax.numpy as jnp
from jax import lax
from jax.experimental import pallas as pl
from jax.experimental.pallas import tpu as pltpu
from jax.experimental.pallas import tpu_sc as plsc

_N = 5000
_NPAD = 5120
_L = 16
_NW = 16
_U = 2
_B = _NW * _L * _U
_NB = _NPAD // _B
_FAR = 2e9
_MAGIC = 0x5AB10000


def _splat_gather(ref, idx_scalar):
    iv = jnp.full((_L,), idx_scalar, jnp.int32)
    return plsc.load_gather(ref, [iv])


def _nms_body(x1h, y1h, x2h, y2h, sch, ordh,
              keep_h, ksc_h, aliveb_h, bidxb_h, knewb_h, flag0_h, flag1_h,
              x1, y1, x2, y2, scu, ordl,
              kx1, ky1, kx2, ky2, kar,
              stat_l, bidx_l, alive_l, keepv, ksc_v,
              stage, knew_l,
              alive_sh, stat_sh, bidx_sh, knew_sh):
    w = lax.axis_index("s")
    cid = lax.axis_index("c")
    lanes = lax.broadcasted_iota(jnp.int32, (_L,), 0)
    lane0 = lanes == 0
    ffalse = lanes < 0
    fone = jnp.full((_L,), 1.0, jnp.float32)
    far = jnp.full((_L,), _FAR, jnp.float32)

    pltpu.sync_copy(x1h, x1)
    pltpu.sync_copy(y1h, y1)
    pltpu.sync_copy(x2h, x2)
    pltpu.sync_copy(y2h, y2)
    pltpu.sync_copy(sch, scu)
    pltpu.sync_copy(ordh, ordl)

    def initk(i, c):
        sl = pl.ds(i * _L, _L)
        kx1[sl] = far
        ky1[sl] = far
        kx2[sl] = far
        ky2[sl] = far
        kar[sl] = fone
        return c
    lax.fori_loop(0, _NPAD // _L, initk, 0)

    @pl.when((cid == 0) & (w == 0))
    def _():
        def initkv(i, c):
            keepv[pl.ds(i * _L, _L)] = jnp.zeros((_L,), jnp.float32)
            ksc_v[pl.ds(i * _L, _L)] = jnp.zeros((_L,), jnp.float32)
            return c
        lax.fori_loop(0, _NPAD // _L, initkv, 0)

    p1 = w * _L
    p2 = (2 * _NW - 1 - w) * _L

    def poll_eq(flag_h, target):
        def cond(c):
            return c != target

        def body(c):
            pltpu.sync_copy(flag_h, knew_l)
            return jnp.max(knew_l[pl.ds(0, _L)])

        lax.while_loop(cond, body, target + 1)

    def put_flag(flag_h, val):
        stage[pl.ds(0, _L)] = jnp.full((_L,), val, jnp.int32)
        pltpu.sync_copy(stage.at[pl.ds(0, _L)], flag_h)

    def load_cands(base):
        orda = ordl[pl.ds(base + p1, _L)]
        ordb = ordl[pl.ds(base + p2, _L)]
        ca = (plsc.load_gather(x1, [orda]), plsc.load_gather(y1, [orda]),
              plsc.load_gather(x2, [orda]), plsc.load_gather(y2, [orda]))
        cb = (plsc.load_gather(x1, [ordb]), plsc.load_gather(y1, [ordb]),
              plsc.load_gather(x2, [ordb]), plsc.load_gather(y2, [ordb]))
        ca = ca + ((ca[2] - ca[0]) * (ca[3] - ca[1]),)
        cb = cb + ((cb[2] - cb[0]) * (cb[3] - cb[1]),)
        return ca, cb

    def make_scan(ca, cb):
        ca_x1, ca_y1, ca_x2, ca_y2, ca_ar = ca
        cb_x1, cb_y1, cb_x2, cb_y2, cb_ar = cb

        def scan_one(t, sa, sb):
            kx1v = _splat_gather(kx1, t)
            ky1v = _splat_gather(ky1, t)
            kx2v = _splat_gather(kx2, t)
            ky2v = _splat_gather(ky2, t)
            karv = _splat_gather(kar, t)

            wa = jnp.maximum(jnp.minimum(ca_x2, kx2v) - jnp.maximum(ca_x1, kx1v), 0.0)
            ha = jnp.maximum(jnp.minimum(ca_y2, ky2v) - jnp.maximum(ca_y1, ky1v), 0.0)
            ia = wa * ha
            sa = sa | (ia > 0.5 * (ca_ar + karv - ia))

            wb = jnp.maximum(jnp.minimum(cb_x2, kx2v) - jnp.maximum(cb_x1, kx1v), 0.0)
            hb = jnp.maximum(jnp.minimum(cb_y2, ky2v) - jnp.maximum(cb_y1, ky1v), 0.0)
            ib = wb * hb
            sb = sb | (ib > 0.5 * (cb_ar + karv - ib))
            return sa, sb

        return scan_one

    def publish_partial_alive(sup_a, sup_b):
        stage[pl.ds(0, _L)] = jnp.where(sup_a, 0, 1).astype(jnp.int32)
        stage[pl.ds(_L, _L)] = jnp.where(sup_b, 0, 1).astype(jnp.int32)
        pltpu.sync_copy(stage.at[pl.ds(0, _L)], alive_sh.at[pl.ds(p1, _L)])
        pltpu.sync_copy(stage.at[pl.ds(_L, _L)], alive_sh.at[pl.ds(p2, _L)])

    def append_deltas(base, k_count, nb, basev):
        def append_delta(t, c):
            idxv = basev + bidx_l[pl.ds(t * _L, _L)]
            m = (lanes + t * _L) < nb
            kvec = lanes + (k_count + t * _L)
            ov = plsc.load_gather(ordl, [idxv])
            gx1 = plsc.load_gather(x1, [ov])
            gy1 = plsc.load_gather(y1, [ov])
            gx2 = plsc.load_gather(x2, [ov])
            gy2 = plsc.load_gather(y2, [ov])
            sv = plsc.load_gather(scu, [ov])
            plsc.store_scatter(ksc_v, [idxv], sv, mask=m)
            plsc.store_scatter(kx1, [kvec], gx1, mask=m)
            plsc.store_scatter(ky1, [kvec], gy1, mask=m)
            plsc.store_scatter(kx2, [kvec], gx2, mask=m)
            plsc.store_scatter(ky2, [kvec], gy2, mask=m)
            plsc.store_scatter(kar, [kvec], (gx2 - gx1) * (gy2 - gy1), mask=m)
            return c

        lax.fori_loop(0, (nb + _L - 1) // _L, append_delta, 0)

    @pl.when(cid == 0)
    def _core0():
        def block_body(jb, k_count):
            base = jb * _B
            basev = jnp.full((_L,), base, jnp.int32)
            ca, cb = load_cands(base)
            scan_one = make_scan(ca, cb)

            def scan_even(t, sup):
                return scan_one(2 * t, *sup)

            sup_a, sup_b = lax.fori_loop(0, (k_count + 1) // 2, scan_even,
                                         (ffalse, ffalse))
            publish_partial_alive(sup_a, sup_b)
            plsc.subcore_barrier()

            @pl.when(w == 0)
            def _():
                poll_eq(flag1_h, _MAGIC + jb + 1)
                pltpu.sync_copy(aliveb_h, stat_l)
                pltpu.sync_copy(alive_sh, alive_l)

                def merge(v, c):
                    sl = pl.ds(v * _L, _L)
                    alive_l[sl] = alive_l[sl] & stat_l[sl]
                    return c
                lax.fori_loop(0, _B // _L, merge, 0)
                pltpu.sync_copy(alive_l, alive_sh)

            plsc.subcore_barrier()

            pltpu.sync_copy(alive_sh, alive_l)
            mypos_a = lanes + p1
            mypos_b = lanes + p2
            ca_x1, ca_y1, ca_x2, ca_y2, ca_ar = ca
            cb_x1, cb_y1, cb_x2, cb_y2, cb_ar = cb

            def a2_scan(cx1, cy1, cx2, cy2, car, mypos):
                def step(t, s2):
                    alive_q = _splat_gather(alive_l, t) != 0
                    qpos = jnp.full((_L,), t, jnp.int32)
                    oq = plsc.load_gather(ordl, [basev + qpos])
                    qx1 = plsc.load_gather(x1, [oq])
                    qy1 = plsc.load_gather(y1, [oq])
                    qx2 = plsc.load_gather(x2, [oq])
                    qy2 = plsc.load_gather(y2, [oq])
                    qar = (qx2 - qx1) * (qy2 - qy1)
                    wv = jnp.maximum(jnp.minimum(cx2, qx2) - jnp.maximum(cx1, qx1), 0.0)
                    hv = jnp.maximum(jnp.minimum(cy2, qy2) - jnp.maximum(cy1, qy1), 0.0)
                    iv = wv * hv
                    return s2 | ((iv > 0.5 * (car + qar - iv)) & alive_q
                                 & (qpos < mypos))
                return step

            sup2_a = lax.fori_loop(0, p1 + _L,
                                   a2_scan(ca_x1, ca_y1, ca_x2, ca_y2, ca_ar,
                                           mypos_a), ffalse)
            sup2_b = lax.fori_loop(0, p2 + _L,
                                   a2_scan(cb_x1, cb_y1, cb_x2, cb_y2, cb_ar,
                                           mypos_b), ffalse)
            av_a = alive_l[pl.ds(p1, _L)] != 0
            av_b = alive_l[pl.ds(p2, _L)] != 0
            stat_a = jnp.where(av_a, jnp.where(sup2_a, 2, 1), 0).astype(jnp.int32)
            stat_b = jnp.where(av_b, jnp.where(sup2_b, 2, 1), 0).astype(jnp.int32)
            stage[pl.ds(0, _L)] = stat_a
            stage[pl.ds(_L, _L)] = stat_b
            pltpu.sync_copy(stage.at[pl.ds(0, _L)], stat_sh.at[pl.ds(p1, _L)])
            pltpu.sync_copy(stage.at[pl.ds(_L, _L)], stat_sh.at[pl.ds(p2, _L)])
            plsc.subcore_barrier()

            @pl.when(w == 0)
            def _():
                pltpu.sync_copy(stat_sh, stat_l)

                def initb(i, c):
                    bidx_l[pl.ds(i * _L, _L)] = jnp.zeros((_L,), jnp.int32)
                    return c
                lax.fori_loop(0, _B // _L, initb, 0)

                def res_vreg(v, c0):
                    sv = stat_l[pl.ds(v * _L, _L)]
                    um = sv == 2

                    def process(c1):
                        def wbody(carry):
                            _, m = carry
                            iv = plsc.all_reduce_ffs(m)
                            m2 = m & (lanes != iv)
                            pv = jnp.full((_L,), v * _L, jnp.int32) + iv
                            gv = basev + pv
                            ov = plsc.load_gather(ordl, [gv])
                            cx1 = plsc.load_gather(x1, [ov])
                            cy1 = plsc.load_gather(y1, [ov])
                            cx2 = plsc.load_gather(x2, [ov])
                            cy2 = plsc.load_gather(y2, [ov])
                            car = (cx2 - cx1) * (cy2 - cy1)

                            def inner(t, sup):
                                sl = pl.ds(base + t * _L, _L)
                                sv2 = stat_l[pl.ds(t * _L, _L)]
                                posv = lanes + t * _L
                                ordv2 = ordl[sl]
                                ex1 = plsc.load_gather(x1, [ordv2])
                                ey1 = plsc.load_gather(y1, [ordv2])
                                ex2 = plsc.load_gather(x2, [ordv2])
                                ey2 = plsc.load_gather(y2, [ordv2])
                                ear = (ex2 - ex1) * (ey2 - ey1)
                                wv = jnp.maximum(jnp.minimum(cx2, ex2)
                                                 - jnp.maximum(cx1, ex1), 0.0)
                                hv = jnp.maximum(jnp.minimum(cy2, ey2)
                                                 - jnp.maximum(cy1, ey1), 0.0)
                                ivr = wv * hv
                                hit = (ivr > 0.5 * (car + ear - ivr))
                                return sup | (hit & (sv2 == 1) & (posv < pv))

                            sup = lax.fori_loop(0, v + 1, inner, ffalse)
                            supi = jnp.any(sup).astype(jnp.int32)
                            val = jnp.full((_L,), 1, jnp.int32) - supi
                            plsc.store_scatter(stat_l, [pv], val, mask=lane0)
                            return (c1, m2)

                        return lax.while_loop(lambda c: jnp.any(c[1]), wbody,
                                              (c1, um))[0]

                    return lax.cond(jnp.any(um), process, lambda c: c, c0)

                lax.fori_loop(0, _B // _L, res_vreg, 0)

                def cmp_vreg(v, nb0):
                    sv = stat_l[pl.ds(v * _L, _L)]
                    km = sv == 1

                    def process(nb1):
                        def wbody(carry):
                            nbc, m = carry
                            iv = plsc.all_reduce_ffs(m)
                            m2 = m & (lanes != iv)
                            pv = jnp.full((_L,), v * _L, jnp.int32) + iv
                            nv = jnp.full((_L,), nbc, jnp.int32)
                            plsc.store_scatter(bidx_l, [nv], pv, mask=lane0)
                            plsc.store_scatter(keepv, [basev + pv], fone,
                                               mask=lane0)
                            return (nbc + 1, m2)

                        return lax.while_loop(lambda c: jnp.any(c[1]), wbody,
                                              (nb1, km))[0]

                    return lax.cond(jnp.any(km), process, lambda n: n, nb0)

                bk = lax.fori_loop(0, _B // _L, cmp_vreg, 0)
                pltpu.sync_copy(bidx_l, bidx_sh)
                pltpu.sync_copy(bidx_l, bidxb_h)
                stage[pl.ds(0, _L)] = jnp.full((_L,), bk, jnp.int32)
                pltpu.sync_copy(stage.at[pl.ds(0, _L)], knew_sh)
                pltpu.sync_copy(stage.at[pl.ds(0, _L)], knewb_h)
                put_flag(flag0_h, _MAGIC + jb + 1)

            plsc.subcore_barrier()

            pltpu.sync_copy(knew_sh, knew_l)
            nb = jnp.max(knew_l[pl.ds(0, _L)])
            pltpu.sync_copy(bidx_sh, bidx_l)
            append_deltas(base, k_count, nb, basev)
            return k_count + nb

        lax.fori_loop(0, _NB, block_body, 0)

        @pl.when(w == 0)
        def _():
            pltpu.sync_copy(keepv, keep_h)
            pltpu.sync_copy(ksc_v, ksc_h)

    @pl.when(cid == 1)
    def _core1():
        def block_body(jb, k_count):
            base = jb * _B
            basev = jnp.full((_L,), base, jnp.int32)
            ca, cb = load_cands(base)
            scan_one = make_scan(ca, cb)

            def scan_odd(t, sup):
                return scan_one(2 * t + 1, *sup)

            sup_a, sup_b = lax.fori_loop(0, k_count // 2, scan_odd,
                                         (ffalse, ffalse))
            publish_partial_alive(sup_a, sup_b)
            plsc.subcore_barrier()

            @pl.when(w == 0)
            def _():
                pltpu.sync_copy(alive_sh, alive_l)
                pltpu.sync_copy(alive_l, aliveb_h)
                put_flag(flag1_h, _MAGIC + jb + 1)
                poll_eq(flag0_h, _MAGIC + jb + 1)

            plsc.subcore_barrier()

            pltpu.sync_copy(knewb_h, knew_l)
            nb = jnp.max(knew_l[pl.ds(0, _L)])
            pltpu.sync_copy(bidxb_h, bidx_l)
            append_deltas(base, k_count, nb, basev)
            return k_count + nb

        lax.fori_loop(0, _NB, block_body, 0)


@jax.jit
def _nms_run(x1u, y1u, x2u, y2u, sc, orderp):
    mesh = plsc.VectorSubcoreMesh(core_axis_name="c", subcore_axis_name="s",
                                  num_cores=2)
    f = pl.kernel(
        _nms_body,
        out_type=(jax.ShapeDtypeStruct((_NPAD,), jnp.float32),
                  jax.ShapeDtypeStruct((_NPAD,), jnp.float32),
                  jax.ShapeDtypeStruct((_B,), jnp.int32),
                  jax.ShapeDtypeStruct((_B,), jnp.int32),
                  jax.ShapeDtypeStruct((_L,), jnp.int32),
                  jax.ShapeDtypeStruct((_L,), jnp.int32),
                  jax.ShapeDtypeStruct((_L,), jnp.int32)),
        mesh=mesh,
        scratch_types=(
            [pltpu.VMEM((_NPAD,), jnp.float32) for _ in range(4)]
            + [pltpu.VMEM((_NPAD,), jnp.float32)]
            + [pltpu.VMEM((_NPAD,), jnp.int32)]
            + [pltpu.VMEM((_NPAD,), jnp.float32) for _ in range(5)]
            + [pltpu.VMEM((_B,), jnp.int32)]
            + [pltpu.VMEM((_B,), jnp.int32)]
            + [pltpu.VMEM((_B,), jnp.int32)]
            + [pltpu.VMEM((_NPAD,), jnp.float32)]
            + [pltpu.VMEM((_NPAD,), jnp.float32)]
            + [pltpu.VMEM((_L * _U,), jnp.int32)]
            + [pltpu.VMEM((_L,), jnp.int32)]
            + [pltpu.VMEM_SHARED((_B,), jnp.int32)]
            + [pltpu.VMEM_SHARED((_B,), jnp.int32)]
            + [pltpu.VMEM_SHARED((_B,), jnp.int32)]
            + [pltpu.VMEM_SHARED((_L,), jnp.int32)]
        ),
        compiler_params=pltpu.CompilerParams(needs_layout_passes=False),
    )
    keep01, ksc = f(x1u, y1u, x2u, y2u, sc, orderp)[:2]
    return keep01, ksc


def kernel(boxes, scores):
    order = jnp.argsort(-scores)
    pad = _NPAD - boxes.shape[0]
    orderp = jnp.concatenate(
        [order, jnp.full((pad,), order[0], order.dtype)], axis=0)
    bp = jnp.pad(boxes, ((0, pad), (0, 0)))
    sp = jnp.pad(scores, (0, pad))
    keep01, ksc = _nms_run(bp[:, 0], bp[:, 1], bp[:, 2], bp[:, 3], sp, orderp)
    keep = keep01[:_N] > 0.5
    kept_scores = ksc[:_N]
    return kept_scores, keep, order
